# Initial kernel scaffold; baseline (speedup 1.0000x reference)
import functools

import jax
import jax.numpy as jnp
from jax import lax
from jax.experimental import pallas as pl
from jax.experimental.pallas import tpu as pltpu
from jax.experimental.pallas import tpu_sc as plsc

ALPHA = 0.1
K = 2

NC = 2
NS = 16
NW = NC * NS
LANES = 16
IDXW = 128
U = 8


def _round_up(x, m):
    return (x + m - 1) // m * m


def _make_deg_kernel(npad, rows_per_tile):
    zrows = npad // NS
    mesh = plsc.VectorSubcoreMesh(core_axis_name="c", subcore_axis_name="s")

    @functools.partial(
        pl.kernel,
        out_type=jax.ShapeDtypeStruct((NC, npad), jnp.float32),
        mesh=mesh,
        scratch_types=[
            pltpu.VMEM_SHARED((npad,), jnp.float32),
            pltpu.VMEM((U, IDXW), jnp.int32),
            pltpu.VMEM((IDXW,), jnp.float32),
            pltpu.VMEM((zrows,), jnp.float32),
            pltpu.SemaphoreType.DMA,
        ],
    )
    def deg_kernel(col_hbm, out_hbm, acc, idx_v, ones_v, zeros_v, sem):
        cid = lax.axis_index("c")
        sid = lax.axis_index("s")
        tid = sid * NC + cid

        for i in range(IDXW // LANES):
            ones_v[pl.ds(i * LANES, LANES)] = jnp.ones((LANES,), jnp.float32)

        def zfill(i, _):
            zeros_v[pl.ds(i * LANES, LANES)] = jnp.zeros((LANES,), jnp.float32)
            return 0

        lax.fori_loop(0, zrows // LANES, zfill, 0)
        pltpu.sync_copy(zeros_v, acc.at[pl.ds(sid * zrows, zrows)])
        plsc.subcore_barrier()

        base = tid * rows_per_tile

        def body(t, _):
            g = base + t * U
            pltpu.sync_copy(col_hbm.at[pl.ds(g, U)], idx_v)
            descs = []
            for u in range(U):
                descs.append(
                    pltpu.async_copy(ones_v, acc.at[idx_v.at[u]], sem, add=True)
                )
            for d in descs:
                d.wait()
            return 0

        lax.fori_loop(0, rows_per_tile // U, body, 0)
        plsc.subcore_barrier()
        pltpu.sync_copy(
            acc.at[pl.ds(sid * zrows, zrows)],
            out_hbm.at[cid, pl.ds(sid * zrows, zrows)],
        )

    return deg_kernel


def _make_prop_kernel(n, npad, rows_per_tile):
    zrows = npad // NS
    zchunk = zrows // 8
    mesh = plsc.VectorSubcoreMesh(core_axis_name="c", subcore_axis_name="s")

    @functools.partial(
        pl.kernel,
        out_type=jax.ShapeDtypeStruct((NC, npad, LANES), jnp.float32),
        mesh=mesh,
        scratch_types=[
            pltpu.VMEM_SHARED((npad, LANES), jnp.float32),
            pltpu.VMEM((U, IDXW), jnp.int32),
            pltpu.VMEM((U, IDXW), jnp.int32),
            pltpu.VMEM((U, IDXW, LANES), jnp.float32),
            pltpu.VMEM((zrows // 8, LANES), jnp.float32),
            pltpu.SemaphoreType.DMA,
            pltpu.SemaphoreType.DMA,
        ],
    )
    def prop_kernel(row_hbm, col_hbm, y_hbm, out_hbm, s_acc, row_v, col_v,
                    msgs_v, zeros_v, gsem, ssem):
        cid = lax.axis_index("c")
        sid = lax.axis_index("s")
        tid = sid * NC + cid

        def zfill(i, _):
            zeros_v[i] = jnp.zeros((LANES,), jnp.float32)
            return 0

        lax.fori_loop(0, zchunk, zfill, 0)
        for k in range(8):
            pltpu.sync_copy(
                zeros_v, s_acc.at[pl.ds(sid * zrows + k * zchunk, zchunk)]
            )
        plsc.subcore_barrier()

        base = tid * rows_per_tile

        def body(t, _):
            g = base + t * U
            pltpu.sync_copy(row_hbm.at[pl.ds(g, U)], row_v)
            pltpu.sync_copy(col_hbm.at[pl.ds(g, U)], col_v)
            gd = []
            for u in range(U):
                gd.append(
                    pltpu.async_copy(y_hbm.at[row_v.at[u]], msgs_v.at[u], gsem)
                )
            for d in gd:
                d.wait()
            sd = []
            for u in range(U):
                sd.append(
                    pltpu.async_copy(
                        msgs_v.at[u], s_acc.at[col_v.at[u]], ssem, add=True
                    )
                )
            for d in sd:
                d.wait()
            return 0

        lax.fori_loop(0, rows_per_tile // U, body, 0)
        plsc.subcore_barrier()
        pltpu.sync_copy(
            s_acc.at[pl.ds(sid * zrows, zrows)],
            out_hbm.at[cid, pl.ds(sid * zrows, zrows)],
        )

    return prop_kernel


def _mlp_body(x_ref, d0_ref, d1_ref, w1_ref, b1_ref, w2_ref, b2_ref, p_ref,
              h_ref, y0_ref, dinv_ref):
    xb = x_ref[...]
    h1 = lax.dot_general(
        xb, w1_ref[...], (((1,), (1,)), ((), ())),
        preferred_element_type=jnp.float32,
    ) + b1_ref[...]
    h1 = jnp.maximum(h1, 0.0) * jnp.clip(p_ref[...], 0.0, 1.0)
    h = lax.dot_general(
        h1, w2_ref[...], (((1,), (1,)), ((), ())),
        preferred_element_type=jnp.float32,
    ) + b2_ref[...]
    deg = d0_ref[...] + d1_ref[...] + 1.0
    dinv = lax.rsqrt(deg)
    h_ref[...] = h
    y0_ref[...] = h * dinv
    dinv_ref[...] = dinv


def _combine_body(s0_ref, s1_ref, y_ref, h_ref, dinv_ref, out_ref, ynext_ref):
    s = s0_ref[...] + s1_ref[...] + y_ref[...]
    out = (1.0 - ALPHA) * dinv_ref[...] * s + ALPHA * h_ref[...]
    out_ref[...] = out
    ynext_ref[...] = out * dinv_ref[...]


def _make_mlp(n, nfeat, nhid, nclass, bn):
    grid = (n // bn,)
    return pl.pallas_call(
        _mlp_body,
        grid=grid,
        in_specs=[
            pl.BlockSpec((bn, nfeat), lambda i: (i, 0)),
            pl.BlockSpec((bn, 1), lambda i: (i, 0)),
            pl.BlockSpec((bn, 1), lambda i: (i, 0)),
            pl.BlockSpec((64, 128), lambda i: (0, 0)),
            pl.BlockSpec((1, 64), lambda i: (0, 0)),
            pl.BlockSpec((16, 64), lambda i: (0, 0)),
            pl.BlockSpec((1, 16), lambda i: (0, 0)),
            pl.BlockSpec((1, 64), lambda i: (0, 0)),
        ],
        out_specs=[
            pl.BlockSpec((bn, nclass), lambda i: (i, 0)),
            pl.BlockSpec((bn, nclass), lambda i: (i, 0)),
            pl.BlockSpec((bn, 1), lambda i: (i, 0)),
        ],
        out_shape=[
            jax.ShapeDtypeStruct((n, nclass), jnp.float32),
            jax.ShapeDtypeStruct((n, nclass), jnp.float32),
            jax.ShapeDtypeStruct((n, 1), jnp.float32),
        ],
    )


def _make_combine(n, nclass, bn):
    grid = (n // bn,)
    spec16 = pl.BlockSpec((bn, nclass), lambda i: (i, 0))
    spec1 = pl.BlockSpec((bn, 1), lambda i: (i, 0))
    return pl.pallas_call(
        _combine_body,
        grid=grid,
        in_specs=[spec16, spec16, spec16, spec16, spec1],
        out_specs=[spec16, spec16],
        out_shape=[
            jax.ShapeDtypeStruct((n, nclass), jnp.float32),
            jax.ShapeDtypeStruct((n, nclass), jnp.float32),
        ],
    )


def kernel(x, edge_index, W1, b1, W2, b2, p):
    n, nfeat = x.shape
    nhid = W1.shape[0]
    nclass = W2.shape[0]
    e = edge_index.shape[1]

    npad = _round_up(n + 1, NS * LANES * 8)
    epad = _round_up(e, NW * IDXW * U)
    rows_per_tile = epad // IDXW // NW

    pad = epad - e
    row = jnp.concatenate([edge_index[0], jnp.zeros((pad,), edge_index.dtype)])
    col = jnp.concatenate([edge_index[1], jnp.full((pad,), n, edge_index.dtype)])
    row2d = row.reshape(epad // IDXW, IDXW)
    col2d = col.reshape(epad // IDXW, IDXW)

    degp = _make_deg_kernel(npad, rows_per_tile)(col2d)
    d0 = degp[0, :n].reshape(n, 1)
    d1 = degp[1, :n].reshape(n, 1)

    bn = 2000
    h, y0, dinv = _make_mlp(n, nfeat, nhid, nclass, bn)(
        x, d0, d1, W1, b1.reshape(1, nhid), W2, b2.reshape(1, nclass),
        p.reshape(1, nhid))

    prop = _make_prop_kernel(n, npad, rows_per_tile)
    combine = _make_combine(n, nclass, bn)

    y = y0
    out = None
    for _ in range(K):
        s_part = prop(row2d, col2d, y)
        out, y = combine(s_part[0, :n], s_part[1, :n], y, h, dinv)
    return out


# speedup vs baseline: 33.0960x; 33.0960x over previous
import functools

import jax
import jax.numpy as jnp
from jax import lax
from jax.experimental import pallas as pl
from jax.experimental.pallas import tpu as pltpu
from jax.experimental.pallas import tpu_sc as plsc

ALPHA = 0.1
K = 2

NC = 2
NS = 16
NW = NC * NS
LANES = 16
IDXW = 128
U = 8


def _round_up(x, m):
    return (x + m - 1) // m * m


def _make_deg_kernel(npad, rows_per_tile):
    zrows = npad // NS
    mesh = plsc.VectorSubcoreMesh(core_axis_name="c", subcore_axis_name="s")

    @functools.partial(
        pl.kernel,
        out_type=jax.ShapeDtypeStruct((NC, npad), jnp.float32),
        mesh=mesh,
        scratch_types=[
            pltpu.VMEM_SHARED((npad,), jnp.float32),
            pltpu.VMEM((U, IDXW), jnp.int32),
            pltpu.VMEM((IDXW,), jnp.float32),
            pltpu.VMEM((zrows,), jnp.float32),
            pltpu.SemaphoreType.DMA,
        ],
        compiler_params=pltpu.CompilerParams(use_tc_tiling_on_sc=False),
    )
    def deg_kernel(col_hbm, out_hbm, acc, idx_v, ones_v, zeros_v, sem):
        cid = lax.axis_index("c")
        sid = lax.axis_index("s")
        tid = sid * NC + cid

        for i in range(IDXW // LANES):
            ones_v[pl.ds(i * LANES, LANES)] = jnp.ones((LANES,), jnp.float32)

        def zfill(i, _):
            zeros_v[pl.ds(i * LANES, LANES)] = jnp.zeros((LANES,), jnp.float32)
            return 0

        lax.fori_loop(0, zrows // LANES, zfill, 0)
        pltpu.sync_copy(zeros_v, acc.at[pl.ds(sid * zrows, zrows)])
        plsc.subcore_barrier()

        base = tid * rows_per_tile

        def body(t, _):
            g = base + t * U
            pltpu.sync_copy(col_hbm.at[pl.ds(g, U)], idx_v)
            descs = []
            for u in range(U):
                descs.append(
                    pltpu.async_copy(ones_v, acc.at[idx_v.at[u]], sem, add=True)
                )
            for d in descs:
                d.wait()
            return 0

        lax.fori_loop(0, rows_per_tile // U, body, 0)
        plsc.subcore_barrier()
        pltpu.sync_copy(
            acc.at[pl.ds(sid * zrows, zrows)],
            out_hbm.at[cid, pl.ds(sid * zrows, zrows)],
        )

    return deg_kernel


def _make_prop_kernel(n, npad, rows_per_tile):
    zrows = npad // NS
    zchunk = zrows // 16
    mesh = plsc.VectorSubcoreMesh(core_axis_name="c", subcore_axis_name="s")

    @functools.partial(
        pl.kernel,
        out_type=jax.ShapeDtypeStruct((NC, npad, LANES), jnp.float32),
        mesh=mesh,
        scratch_types=[
            pltpu.VMEM_SHARED((npad, LANES), jnp.float32),
            pltpu.VMEM((U, IDXW), jnp.int32),
            pltpu.VMEM((U, IDXW), jnp.int32),
            pltpu.VMEM((U, IDXW, LANES), jnp.float32),
            pltpu.VMEM((zrows // 16, LANES), jnp.float32),
            pltpu.SemaphoreType.DMA,
            pltpu.SemaphoreType.DMA,
        ],
        compiler_params=pltpu.CompilerParams(use_tc_tiling_on_sc=False),
    )
    def prop_kernel(row_hbm, col_hbm, y_hbm, out_hbm, s_acc, row_v, col_v,
                    msgs_v, zeros_v, gsem, ssem):
        cid = lax.axis_index("c")
        sid = lax.axis_index("s")
        tid = sid * NC + cid

        def zfill(i, _):
            zeros_v[i] = jnp.zeros((LANES,), jnp.float32)
            return 0

        lax.fori_loop(0, zchunk, zfill, 0)
        for k in range(16):
            pltpu.sync_copy(
                zeros_v, s_acc.at[pl.ds(sid * zrows + k * zchunk, zchunk)]
            )
        plsc.subcore_barrier()

        base = tid * rows_per_tile

        def body(t, _):
            g = base + t * U
            pltpu.sync_copy(row_hbm.at[pl.ds(g, U)], row_v)
            pltpu.sync_copy(col_hbm.at[pl.ds(g, U)], col_v)
            gd = []
            for u in range(U):
                gd.append(
                    pltpu.async_copy(y_hbm.at[row_v.at[u]], msgs_v.at[u], gsem)
                )
            for d in gd:
                d.wait()
            sd = []
            for u in range(U):
                sd.append(
                    pltpu.async_copy(
                        msgs_v.at[u], s_acc.at[col_v.at[u]], ssem, add=True
                    )
                )
            for d in sd:
                d.wait()
            return 0

        lax.fori_loop(0, rows_per_tile // U, body, 0)
        plsc.subcore_barrier()
        pltpu.sync_copy(
            s_acc.at[pl.ds(sid * zrows, zrows)],
            out_hbm.at[cid, pl.ds(sid * zrows, zrows)],
        )

    return prop_kernel


def _mlp_body(x_ref, d0_ref, d1_ref, w1_ref, b1_ref, w2_ref, b2_ref, p_ref,
              h_ref, y0_ref, dinv_ref):
    xb = x_ref[...]
    h1 = lax.dot_general(
        xb, w1_ref[...], (((1,), (1,)), ((), ())),
        preferred_element_type=jnp.float32,
    ) + b1_ref[...]
    h1 = jnp.maximum(h1, 0.0) * jnp.clip(p_ref[...], 0.0, 1.0)
    h = lax.dot_general(
        h1, w2_ref[...], (((1,), (1,)), ((), ())),
        preferred_element_type=jnp.float32,
    ) + b2_ref[...]
    deg = d0_ref[...] + d1_ref[...] + 1.0
    dinv = lax.rsqrt(deg)
    h_ref[...] = h
    y0_ref[...] = h * dinv
    dinv_ref[...] = dinv


def _combine_body(s0_ref, s1_ref, y_ref, h_ref, dinv_ref, out_ref, ynext_ref):
    s = s0_ref[...] + s1_ref[...] + y_ref[...]
    out = (1.0 - ALPHA) * dinv_ref[...] * s + ALPHA * h_ref[...]
    out_ref[...] = out
    ynext_ref[...] = out * dinv_ref[...]


def _make_mlp(n, nfeat, nhid, nclass, bn):
    grid = (n // bn,)
    return pl.pallas_call(
        _mlp_body,
        grid=grid,
        in_specs=[
            pl.BlockSpec((bn, nfeat), lambda i: (i, 0)),
            pl.BlockSpec((bn, 1), lambda i: (i, 0)),
            pl.BlockSpec((bn, 1), lambda i: (i, 0)),
            pl.BlockSpec((64, 128), lambda i: (0, 0)),
            pl.BlockSpec((1, 64), lambda i: (0, 0)),
            pl.BlockSpec((16, 64), lambda i: (0, 0)),
            pl.BlockSpec((1, 16), lambda i: (0, 0)),
            pl.BlockSpec((1, 64), lambda i: (0, 0)),
        ],
        out_specs=[
            pl.BlockSpec((bn, nclass), lambda i: (i, 0)),
            pl.BlockSpec((bn, nclass), lambda i: (i, 0)),
            pl.BlockSpec((bn, 1), lambda i: (i, 0)),
        ],
        out_shape=[
            jax.ShapeDtypeStruct((n, nclass), jnp.float32),
            jax.ShapeDtypeStruct((n, nclass), jnp.float32),
            jax.ShapeDtypeStruct((n, 1), jnp.float32),
        ],
    )


def _make_combine(n, nclass, bn):
    grid = (n // bn,)
    spec16 = pl.BlockSpec((bn, nclass), lambda i: (i, 0))
    spec1 = pl.BlockSpec((bn, 1), lambda i: (i, 0))
    return pl.pallas_call(
        _combine_body,
        grid=grid,
        in_specs=[spec16, spec16, spec16, spec16, spec1],
        out_specs=[spec16, spec16],
        out_shape=[
            jax.ShapeDtypeStruct((n, nclass), jnp.float32),
            jax.ShapeDtypeStruct((n, nclass), jnp.float32),
        ],
    )


def kernel(x, edge_index, W1, b1, W2, b2, p):
    n, nfeat = x.shape
    nhid = W1.shape[0]
    nclass = W2.shape[0]
    e = edge_index.shape[1]

    npad = _round_up(n + 1, NS * LANES * 8)
    epad = _round_up(e, NW * IDXW * U)
    rows_per_tile = epad // IDXW // NW

    pad = epad - e
    row = jnp.concatenate([edge_index[0], jnp.zeros((pad,), edge_index.dtype)])
    col = jnp.concatenate([edge_index[1], jnp.full((pad,), n, edge_index.dtype)])
    row2d = row.reshape(epad // IDXW, IDXW)
    col2d = col.reshape(epad // IDXW, IDXW)

    degp = _make_deg_kernel(npad, rows_per_tile)(col2d)
    d0 = degp[0, :n].reshape(n, 1)
    d1 = degp[1, :n].reshape(n, 1)

    bn = 2000
    h, y0, dinv = _make_mlp(n, nfeat, nhid, nclass, bn)(
        x, d0, d1, W1, b1.reshape(1, nhid), W2, b2.reshape(1, nclass),
        p.reshape(1, nhid))

    prop = _make_prop_kernel(n, npad, rows_per_tile)
    combine = _make_combine(n, nclass, bn)

    y = y0
    out = None
    for _ in range(K):
        s_part = prop(row2d, col2d, y)
        out, y = combine(s_part[0, :n], s_part[1, :n], y, h, dinv)
    return out


# speedup vs baseline: 35.0505x; 1.0591x over previous
import functools

import jax
import jax.numpy as jnp
from jax import lax
from jax.experimental import pallas as pl
from jax.experimental.pallas import tpu as pltpu
from jax.experimental.pallas import tpu_sc as plsc

ALPHA = 0.1
K = 2

NC = 2
NS = 16
NW = NC * NS
LANES = 16
IDXW = 128
UP = 6


def _round_up(x, m):
    return (x + m - 1) // m * m


def _make_deg_kernel(npad, rows_per_tile):
    zrows = npad // NS
    mesh = plsc.VectorSubcoreMesh(core_axis_name="c", subcore_axis_name="s")

    @functools.partial(
        pl.kernel,
        out_type=jax.ShapeDtypeStruct((NC, npad), jnp.float32),
        mesh=mesh,
        scratch_types=[
            pltpu.VMEM_SHARED((npad,), jnp.float32),
            pltpu.VMEM((UP, IDXW), jnp.int32),
            pltpu.VMEM((IDXW,), jnp.float32),
            pltpu.VMEM((zrows,), jnp.float32),
            pltpu.SemaphoreType.DMA,
        ],
        compiler_params=pltpu.CompilerParams(use_tc_tiling_on_sc=False),
    )
    def deg_kernel(col_hbm, out_hbm, acc, idx_v, ones_v, zeros_v, sem):
        cid = lax.axis_index("c")
        sid = lax.axis_index("s")
        tid = sid * NC + cid

        for i in range(IDXW // LANES):
            ones_v[pl.ds(i * LANES, LANES)] = jnp.ones((LANES,), jnp.float32)

        def zfill(i, _):
            zeros_v[pl.ds(i * LANES, LANES)] = jnp.zeros((LANES,), jnp.float32)
            return 0

        lax.fori_loop(0, zrows // LANES, zfill, 0)
        pltpu.sync_copy(zeros_v, acc.at[pl.ds(sid * zrows, zrows)])
        plsc.subcore_barrier()

        base = tid * rows_per_tile

        def body(t, _):
            g = base + t * UP
            pltpu.sync_copy(col_hbm.at[pl.ds(g, UP)], idx_v)
            descs = []
            for u in range(UP):
                descs.append(
                    pltpu.async_copy(ones_v, acc.at[idx_v.at[u]], sem, add=True)
                )
            for d in descs:
                d.wait()
            return 0

        lax.fori_loop(0, rows_per_tile // UP, body, 0)
        plsc.subcore_barrier()
        pltpu.sync_copy(
            acc.at[pl.ds(sid * zrows, zrows)],
            out_hbm.at[cid, pl.ds(sid * zrows, zrows)],
        )

    return deg_kernel


def _make_prop_kernel(n, npad, rows_per_tile, total_rows):
    zrows = npad // NS
    ngroups = rows_per_tile // UP
    assert ngroups % 2 == 0
    mesh = plsc.VectorSubcoreMesh(core_axis_name="c", subcore_axis_name="s")

    @functools.partial(
        pl.kernel,
        out_type=jax.ShapeDtypeStruct((NC, npad, LANES), jnp.float32),
        mesh=mesh,
        scratch_types=[
            pltpu.VMEM_SHARED((npad, LANES), jnp.float32),
            pltpu.VMEM((2, UP, IDXW), jnp.int32),
            pltpu.VMEM((2, UP, IDXW), jnp.int32),
            pltpu.VMEM((2, UP, IDXW, LANES), jnp.float32),
            pltpu.VMEM((IDXW, LANES), jnp.float32),
            pltpu.SemaphoreType.DMA,
            pltpu.SemaphoreType.DMA,
            pltpu.SemaphoreType.DMA,
            pltpu.SemaphoreType.DMA,
        ],
        compiler_params=pltpu.CompilerParams(use_tc_tiling_on_sc=False),
    )
    def prop_kernel(row_hbm, col_hbm, y_hbm, out_hbm, s_acc, row_v, col_v,
                    msgs_v, zbuf, gsem0, gsem1, ssem0, ssem1):
        cid = lax.axis_index("c")
        sid = lax.axis_index("s")
        tid = sid * NC + cid
        gsems = (gsem0, gsem1)
        ssems = (ssem0, ssem1)

        def zfill(i, _):
            zbuf[i] = jnp.zeros((LANES,), jnp.float32)
            return 0

        lax.fori_loop(0, IDXW, zfill, 0)
        for k in range(zrows // IDXW):
            pltpu.sync_copy(
                zbuf, s_acc.at[pl.ds(sid * zrows + k * IDXW, IDXW)]
            )
        plsc.subcore_barrier()

        base = tid * rows_per_tile
        gmax = total_rows - UP

        def load_fire(g, p):
            pltpu.sync_copy(row_hbm.at[pl.ds(g, UP)], row_v.at[p])
            pltpu.sync_copy(col_hbm.at[pl.ds(g, UP)], col_v.at[p])
            for u in range(UP):
                pltpu.async_copy(y_hbm.at[row_v.at[p, u]], msgs_v.at[p, u],
                                 gsems[p])

        def drain_g(p):
            for u in range(UP):
                pltpu.make_async_copy(
                    y_hbm.at[row_v.at[p, u]], msgs_v.at[p, u], gsems[p]
                ).wait()

        def fire_scat(p):
            for u in range(UP):
                pltpu.async_copy(msgs_v.at[p, u], s_acc.at[col_v.at[p, u]],
                                 ssems[p], add=True)

        def drain_s(p):
            for u in range(UP):
                pltpu.make_async_copy(
                    y_hbm.at[pl.ds(0, IDXW)], msgs_v.at[p, u], ssems[p]
                ).wait()

        load_fire(base, 0)

        def body(tp, _):
            b = base + (2 * tp + 1) * UP
            c = jnp.minimum(base + (2 * tp + 2) * UP, gmax)
            load_fire(b, 1)
            drain_g(0)
            fire_scat(0)
            drain_g(1)
            fire_scat(1)
            drain_s(0)
            load_fire(c, 0)
            drain_s(1)
            return 0

        lax.fori_loop(0, ngroups // 2, body, 0)
        drain_g(0)
        plsc.subcore_barrier()
        pltpu.sync_copy(
            s_acc.at[pl.ds(sid * zrows, zrows)],
            out_hbm.at[cid, pl.ds(sid * zrows, zrows)],
        )

    return prop_kernel


def _mlp_body(x_ref, w1_ref, b1_ref, w2_ref, b2_ref, p_ref, h_ref):
    h1 = lax.dot_general(
        x_ref[...], w1_ref[...], (((1,), (1,)), ((), ())),
        preferred_element_type=jnp.float32,
    ) + b1_ref[...]
    h1 = jnp.maximum(h1, 0.0) * jnp.clip(p_ref[...], 0.0, 1.0)
    h_ref[...] = lax.dot_general(
        h1, w2_ref[...], (((1,), (1,)), ((), ())),
        preferred_element_type=jnp.float32,
    ) + b2_ref[...]


def _prep_body(d0_ref, d1_ref, h_ref, dinv_ref, y0_ref):
    dinv = lax.rsqrt(d0_ref[...] + d1_ref[...] + 1.0)
    dinv_ref[...] = dinv
    y0_ref[...] = h_ref[...] * dinv


def _make_mlp(n, nfeat, nhid, nclass, bn):
    return pl.pallas_call(
        _mlp_body,
        grid=(n // bn,),
        in_specs=[
            pl.BlockSpec((bn, nfeat), lambda i: (i, 0)),
            pl.BlockSpec((nhid, nfeat), lambda i: (0, 0)),
            pl.BlockSpec((1, nhid), lambda i: (0, 0)),
            pl.BlockSpec((nclass, nhid), lambda i: (0, 0)),
            pl.BlockSpec((1, nclass), lambda i: (0, 0)),
            pl.BlockSpec((1, nhid), lambda i: (0, 0)),
        ],
        out_specs=pl.BlockSpec((bn, nclass), lambda i: (i, 0)),
        out_shape=jax.ShapeDtypeStruct((n, nclass), jnp.float32),
    )


def _make_prep(n, nclass, bn):
    spec16 = pl.BlockSpec((bn, nclass), lambda i: (i, 0))
    spec1 = pl.BlockSpec((bn, 1), lambda i: (i, 0))
    return pl.pallas_call(
        _prep_body,
        grid=(n // bn,),
        in_specs=[spec1, spec1, spec16],
        out_specs=[spec1, spec16],
        out_shape=[
            jax.ShapeDtypeStruct((n, 1), jnp.float32),
            jax.ShapeDtypeStruct((n, nclass), jnp.float32),
        ],
    )


def _make_combine(n, npad, nclass, bn, want_out, want_y):
    spec16 = pl.BlockSpec((bn, nclass), lambda i: (i, 0))
    spec1 = pl.BlockSpec((bn, 1), lambda i: (i, 0))
    spart0 = pl.BlockSpec((1, bn, nclass), lambda i: (0, i, 0))
    spart1 = pl.BlockSpec((1, bn, nclass), lambda i: (1, i, 0))

    def body(sp_ref, sp1_ref, y_ref, h_ref, dinv_ref, *outs):
        s = sp_ref[0] + sp1_ref[0] + y_ref[...]
        out = (1.0 - ALPHA) * dinv_ref[...] * s + ALPHA * h_ref[...]
        i = 0
        if want_out:
            outs[i][...] = out
            i += 1
        if want_y:
            outs[i][...] = out * dinv_ref[...]

    nouts = int(want_out) + int(want_y)
    return pl.pallas_call(
        body,
        grid=(n // bn,),
        in_specs=[spart0, spart1, spec16, spec16, spec1],
        out_specs=[spec16] * nouts,
        out_shape=[jax.ShapeDtypeStruct((n, nclass), jnp.float32)] * nouts,
    )


def kernel(x, edge_index, W1, b1, W2, b2, p):
    n, nfeat = x.shape
    nhid = W1.shape[0]
    nclass = W2.shape[0]
    e = edge_index.shape[1]

    npad = _round_up(n + 1, NS * LANES * 8)
    epad = _round_up(e, NW * IDXW * UP * 2)
    rows_per_tile = epad // IDXW // NW
    total_rows = epad // IDXW

    pad = epad - e
    row = jnp.concatenate([edge_index[0], jnp.zeros((pad,), edge_index.dtype)])
    col = jnp.concatenate([edge_index[1], jnp.full((pad,), n, edge_index.dtype)])
    row2d = row.reshape(total_rows, IDXW)
    col2d = col.reshape(total_rows, IDXW)

    degp = _make_deg_kernel(npad, rows_per_tile)(col2d)

    bn = 2000
    h = _make_mlp(n, nfeat, nhid, nclass, bn)(
        x, W1, b1.reshape(1, nhid), W2, b2.reshape(1, nclass),
        p.reshape(1, nhid))

    d0 = degp[0, :n].reshape(n, 1)
    d1 = degp[1, :n].reshape(n, 1)
    dinv, y0 = _make_prep(n, nclass, bn)(d0, d1, h)

    prop = _make_prop_kernel(n, npad, rows_per_tile, total_rows)
    combine_mid = _make_combine(n, npad, nclass, bn, want_out=False, want_y=True)
    combine_fin = _make_combine(n, npad, nclass, bn, want_out=True, want_y=False)

    s_part = prop(row2d, col2d, y0)
    (y1,) = combine_mid(s_part, s_part, y0, h, dinv)
    s_part2 = prop(row2d, col2d, y1)
    (out,) = combine_fin(s_part2, s_part2, y1, h, dinv)
    return out


# speedup vs baseline: 42.3636x; 1.2086x over previous
import functools

import jax
import jax.numpy as jnp
from jax import lax
from jax.experimental import pallas as pl
from jax.experimental.pallas import tpu as pltpu
from jax.experimental.pallas import tpu_sc as plsc

ALPHA = 0.1
K = 2

NC = 2
NS = 16
NW = NC * NS
LANES = 16
IDXW = 128
UP = 6


def _round_up(x, m):
    return (x + m - 1) // m * m


FRAC_C0 = 0.37


def _tile_quota(cid, sid, total_rows, frac0):
    q0 = int(round(total_rows * frac0))
    qc = jnp.where(cid == 0, q0, total_rows - q0)
    base_c = jnp.where(cid == 0, 0, q0)
    per = qc // NS
    ext = qc % NS
    q = per + jnp.where(sid < ext, 1, 0)
    start = base_c + sid * per + jnp.minimum(sid, ext)
    return start, q


def _make_deg_kernel(npad, total_rows):
    zrows = npad // NS
    mesh = plsc.VectorSubcoreMesh(core_axis_name="c", subcore_axis_name="s")

    @functools.partial(
        pl.kernel,
        out_type=jax.ShapeDtypeStruct((NC, npad), jnp.float32),
        mesh=mesh,
        scratch_types=[
            pltpu.VMEM_SHARED((npad,), jnp.float32),
            pltpu.VMEM((UP, IDXW), jnp.int32),
            pltpu.VMEM((IDXW,), jnp.float32),
            pltpu.VMEM((zrows,), jnp.float32),
            pltpu.SemaphoreType.DMA,
        ],
        compiler_params=pltpu.CompilerParams(use_tc_tiling_on_sc=False),
    )
    def deg_kernel(edge_hbm, out_hbm, acc, idx_v, ones_v, zeros_v, sem):
        cid = lax.axis_index("c")
        sid = lax.axis_index("s")
        start, q = _tile_quota(cid, sid, total_rows, 0.5)

        for i in range(IDXW // LANES):
            ones_v[pl.ds(i * LANES, LANES)] = jnp.ones((LANES,), jnp.float32)

        def zfill(i, _):
            zeros_v[pl.ds(i * LANES, LANES)] = jnp.zeros((LANES,), jnp.float32)
            return 0

        lax.fori_loop(0, zrows // LANES, zfill, 0)
        pltpu.sync_copy(zeros_v, acc.at[pl.ds(sid * zrows, zrows)])
        plsc.subcore_barrier()

        def body(t, _):
            g = start + t * UP
            pltpu.sync_copy(edge_hbm.at[1, pl.ds(g, UP)], idx_v)
            descs = []
            for u in range(UP):
                descs.append(
                    pltpu.async_copy(ones_v, acc.at[idx_v.at[u]], sem, add=True)
                )
            for d in descs:
                d.wait()
            return 0

        ng = q // UP
        lax.fori_loop(0, ng, body, 0)

        def tail(i, _):
            g = start + ng * UP + i
            pltpu.sync_copy(edge_hbm.at[1, pl.ds(g, 1)], idx_v.at[pl.ds(0, 1)])
            pltpu.sync_copy(ones_v, acc.at[idx_v.at[0]], add=True)
            return 0

        lax.fori_loop(0, q - ng * UP, tail, 0)
        plsc.subcore_barrier()
        pltpu.sync_copy(
            acc.at[pl.ds(sid * zrows, zrows)],
            out_hbm.at[cid, pl.ds(sid * zrows, zrows)],
        )

    return deg_kernel


def _make_prop_kernel(n, npad, total_rows):
    zrows = npad // NS
    mesh = plsc.VectorSubcoreMesh(core_axis_name="c", subcore_axis_name="s")

    @functools.partial(
        pl.kernel,
        out_type=jax.ShapeDtypeStruct((NC, npad, LANES), jnp.float32),
        mesh=mesh,
        scratch_types=[
            pltpu.VMEM_SHARED((npad, LANES), jnp.float32),
            pltpu.VMEM((2, UP, IDXW), jnp.int32),
            pltpu.VMEM((2, UP, IDXW), jnp.int32),
            pltpu.VMEM((2, UP, IDXW, LANES), jnp.float32),
            pltpu.VMEM((IDXW, LANES), jnp.float32),
            pltpu.SemaphoreType.DMA,
            pltpu.SemaphoreType.DMA,
            pltpu.SemaphoreType.DMA,
            pltpu.SemaphoreType.DMA,
        ],
        compiler_params=pltpu.CompilerParams(use_tc_tiling_on_sc=False),
    )
    def prop_kernel(edge_hbm, y_hbm, out_hbm, s_acc, row_v, col_v,
                    msgs_v, zbuf, gsem0, gsem1, ssem0, ssem1):
        cid = lax.axis_index("c")
        sid = lax.axis_index("s")
        start, q = _tile_quota(cid, sid, total_rows, FRAC_C0)
        gsems = (gsem0, gsem1)
        ssems = (ssem0, ssem1)

        def zfill(i, _):
            zbuf[i] = jnp.zeros((LANES,), jnp.float32)
            return 0

        lax.fori_loop(0, IDXW, zfill, 0)
        for k in range(zrows // IDXW):
            pltpu.sync_copy(
                zbuf, s_acc.at[pl.ds(sid * zrows + k * IDXW, IDXW)]
            )
        plsc.subcore_barrier()

        gmax = total_rows - UP

        def load_fire(g, p):
            pltpu.sync_copy(edge_hbm.at[0, pl.ds(g, UP)], row_v.at[p])
            pltpu.sync_copy(edge_hbm.at[1, pl.ds(g, UP)], col_v.at[p])
            for u in range(UP):
                pltpu.async_copy(y_hbm.at[row_v.at[p, u]], msgs_v.at[p, u],
                                 gsems[p])

        def drain_g(p):
            for u in range(UP):
                pltpu.make_async_copy(
                    y_hbm.at[row_v.at[p, u]], msgs_v.at[p, u], gsems[p]
                ).wait()

        def fire_scat(p):
            for u in range(UP):
                pltpu.async_copy(msgs_v.at[p, u], s_acc.at[col_v.at[p, u]],
                                 ssems[p], add=True)

        def drain_s(p):
            for u in range(UP):
                pltpu.make_async_copy(
                    y_hbm.at[pl.ds(0, IDXW)], msgs_v.at[p, u], ssems[p]
                ).wait()

        npairs = q // (2 * UP)
        rem = q - npairs * 2 * UP
        tg = rem // UP
        tr = rem - tg * UP

        load_fire(start, 0)

        def body(tp, _):
            b = start + (2 * tp + 1) * UP
            c = jnp.minimum(start + (2 * tp + 2) * UP, gmax)
            load_fire(b, 1)
            drain_g(0)
            fire_scat(0)
            drain_g(1)
            fire_scat(1)
            drain_s(0)
            load_fire(c, 0)
            drain_s(1)
            return 0

        lax.fori_loop(0, npairs, body, 0)
        drain_g(0)

        @pl.when(tg == 1)
        def _():
            fire_scat(0)
            drain_s(0)

        def tail(i, _):
            g = start + npairs * 2 * UP + tg * UP + i
            pltpu.sync_copy(edge_hbm.at[0, pl.ds(g, 1)],
                            row_v.at[1, pl.ds(0, 1)])
            pltpu.sync_copy(edge_hbm.at[1, pl.ds(g, 1)],
                            col_v.at[1, pl.ds(0, 1)])
            pltpu.async_copy(y_hbm.at[row_v.at[1, 0]], msgs_v.at[1, 0],
                             gsem1).wait()
            pltpu.sync_copy(msgs_v.at[1, 0], s_acc.at[col_v.at[1, 0]],
                            add=True)
            return 0

        lax.fori_loop(0, tr, tail, 0)
        plsc.subcore_barrier()
        pltpu.sync_copy(
            s_acc.at[pl.ds(sid * zrows, zrows)],
            out_hbm.at[cid, pl.ds(sid * zrows, zrows)],
        )

    return prop_kernel


def _mlp_body(x_ref, w1_ref, b1_ref, w2_ref, b2_ref, p_ref, h_ref):
    h1 = lax.dot_general(
        x_ref[...], w1_ref[...], (((1,), (1,)), ((), ())),
        preferred_element_type=jnp.float32,
    ) + b1_ref[...]
    h1 = jnp.maximum(h1, 0.0) * jnp.clip(p_ref[...], 0.0, 1.0)
    h_ref[...] = lax.dot_general(
        h1, w2_ref[...], (((1,), (1,)), ((), ())),
        preferred_element_type=jnp.float32,
    ) + b2_ref[...]


def _prep_body(d0_ref, d1_ref, h_ref, dinv_ref, y0_ref):
    dinv = lax.rsqrt(d0_ref[...] + d1_ref[...] + 1.0)
    dinv_ref[...] = dinv
    y0_ref[...] = h_ref[...] * dinv


def _make_mlp(n, nfeat, nhid, nclass, bn):
    return pl.pallas_call(
        _mlp_body,
        grid=(n // bn,),
        in_specs=[
            pl.BlockSpec((bn, nfeat), lambda i: (i, 0)),
            pl.BlockSpec((nhid, nfeat), lambda i: (0, 0)),
            pl.BlockSpec((1, nhid), lambda i: (0, 0)),
            pl.BlockSpec((nclass, nhid), lambda i: (0, 0)),
            pl.BlockSpec((1, nclass), lambda i: (0, 0)),
            pl.BlockSpec((1, nhid), lambda i: (0, 0)),
        ],
        out_specs=pl.BlockSpec((bn, nclass), lambda i: (i, 0)),
        out_shape=jax.ShapeDtypeStruct((n, nclass), jnp.float32),
    )


def _make_prep(n, nclass, bn):
    spec16 = pl.BlockSpec((bn, nclass), lambda i: (i, 0))
    spec1 = pl.BlockSpec((bn, 1), lambda i: (i, 0))
    return pl.pallas_call(
        _prep_body,
        grid=(n // bn,),
        in_specs=[spec1, spec1, spec16],
        out_specs=[spec1, spec16],
        out_shape=[
            jax.ShapeDtypeStruct((n, 1), jnp.float32),
            jax.ShapeDtypeStruct((n, nclass), jnp.float32),
        ],
    )


def _make_combine(n, npad, nclass, bn, want_out, want_y):
    spec16 = pl.BlockSpec((bn, nclass), lambda i: (i, 0))
    spec1 = pl.BlockSpec((bn, 1), lambda i: (i, 0))
    spart0 = pl.BlockSpec((1, bn, nclass), lambda i: (0, i, 0))
    spart1 = pl.BlockSpec((1, bn, nclass), lambda i: (1, i, 0))

    def body(sp_ref, sp1_ref, y_ref, h_ref, dinv_ref, *outs):
        s = sp_ref[0] + sp1_ref[0] + y_ref[...]
        out = (1.0 - ALPHA) * dinv_ref[...] * s + ALPHA * h_ref[...]
        i = 0
        if want_out:
            outs[i][...] = out
            i += 1
        if want_y:
            outs[i][...] = out * dinv_ref[...]

    nouts = int(want_out) + int(want_y)
    return pl.pallas_call(
        body,
        grid=(n // bn,),
        in_specs=[spart0, spart1, spec16, spec16, spec1],
        out_specs=[spec16] * nouts,
        out_shape=[jax.ShapeDtypeStruct((n, nclass), jnp.float32)] * nouts,
    )


def kernel(x, edge_index, W1, b1, W2, b2, p):
    n, nfeat = x.shape
    nhid = W1.shape[0]
    nclass = W2.shape[0]
    e = edge_index.shape[1]

    npad = _round_up(n + 1, NS * LANES * 8)
    er = _round_up(e, IDXW)
    if er != e:
        fill = jnp.stack([
            jnp.zeros((er - e,), edge_index.dtype),
            jnp.full((er - e,), n, edge_index.dtype),
        ])
        edge_index = jnp.concatenate([edge_index, fill], axis=1)
    total_rows = er // IDXW
    edge3d = edge_index.reshape(2, total_rows, IDXW)

    degp = _make_deg_kernel(npad, total_rows)(edge3d)

    bn = 2000
    h = _make_mlp(n, nfeat, nhid, nclass, bn)(
        x, W1, b1.reshape(1, nhid), W2, b2.reshape(1, nclass),
        p.reshape(1, nhid))

    d0 = degp[0, :n].reshape(n, 1)
    d1 = degp[1, :n].reshape(n, 1)
    dinv, y0 = _make_prep(n, nclass, bn)(d0, d1, h)

    prop = _make_prop_kernel(n, npad, total_rows)
    combine_mid = _make_combine(n, npad, nclass, bn, want_out=False, want_y=True)
    combine_fin = _make_combine(n, npad, nclass, bn, want_out=True, want_y=False)

    s_part = prop(edge3d, y0)
    (y1,) = combine_mid(s_part, s_part, y0, h, dinv)
    s_part2 = prop(edge3d, y1)
    (out,) = combine_fin(s_part2, s_part2, y1, h, dinv)
    return out


# speedup vs baseline: 44.6062x; 1.0529x over previous
import functools

import jax
import jax.numpy as jnp
from jax import lax
from jax.experimental import pallas as pl
from jax.experimental.pallas import tpu as pltpu
from jax.experimental.pallas import tpu_sc as plsc

ALPHA = 0.1
K = 2

NC = 2
NS = 16
NW = NC * NS
LANES = 16
IDXW = 128
UP = 6


def _round_up(x, m):
    return (x + m - 1) // m * m


FRAC_C0 = 0.47


def _tile_quota(cid, sid, total_rows, frac0):
    q0 = int(round(total_rows * frac0))
    qc = jnp.where(cid == 0, q0, total_rows - q0)
    base_c = jnp.where(cid == 0, 0, q0)
    per = qc // NS
    ext = qc % NS
    q = per + jnp.where(sid < ext, 1, 0)
    start = base_c + sid * per + jnp.minimum(sid, ext)
    return start, q


def _make_deg_kernel(npad, total_rows):
    zrows = npad // NS
    mesh = plsc.VectorSubcoreMesh(core_axis_name="c", subcore_axis_name="s")

    @functools.partial(
        pl.kernel,
        out_type=jax.ShapeDtypeStruct((NC, npad), jnp.float32),
        mesh=mesh,
        scratch_types=[
            pltpu.VMEM_SHARED((npad,), jnp.float32),
            pltpu.VMEM((UP, IDXW), jnp.int32),
            pltpu.VMEM((IDXW,), jnp.float32),
            pltpu.VMEM((zrows,), jnp.float32),
            pltpu.SemaphoreType.DMA,
        ],
        compiler_params=pltpu.CompilerParams(use_tc_tiling_on_sc=False),
    )
    def deg_kernel(edge_hbm, out_hbm, acc, idx_v, ones_v, zeros_v, sem):
        cid = lax.axis_index("c")
        sid = lax.axis_index("s")
        start, q = _tile_quota(cid, sid, total_rows, 0.5)

        for i in range(IDXW // LANES):
            ones_v[pl.ds(i * LANES, LANES)] = jnp.ones((LANES,), jnp.float32)

        def zfill(i, _):
            zeros_v[pl.ds(i * LANES, LANES)] = jnp.zeros((LANES,), jnp.float32)
            return 0

        lax.fori_loop(0, zrows // LANES, zfill, 0)
        pltpu.sync_copy(zeros_v, acc.at[pl.ds(sid * zrows, zrows)])
        plsc.subcore_barrier()

        def body(t, _):
            g = start + t * UP
            pltpu.sync_copy(edge_hbm.at[1, pl.ds(g, UP)], idx_v)
            descs = []
            for u in range(UP):
                descs.append(
                    pltpu.async_copy(ones_v, acc.at[idx_v.at[u]], sem, add=True)
                )
            for d in descs:
                d.wait()
            return 0

        ng = q // UP
        lax.fori_loop(0, ng, body, 0)

        def tail(i, _):
            g = start + ng * UP + i
            pltpu.sync_copy(edge_hbm.at[1, pl.ds(g, 1)], idx_v.at[pl.ds(0, 1)])
            pltpu.sync_copy(ones_v, acc.at[idx_v.at[0]], add=True)
            return 0

        lax.fori_loop(0, q - ng * UP, tail, 0)
        plsc.subcore_barrier()
        pltpu.sync_copy(
            acc.at[pl.ds(sid * zrows, zrows)],
            out_hbm.at[cid, pl.ds(sid * zrows, zrows)],
        )

    return deg_kernel


def _make_prop_kernel(n, npad, total_rows):
    zrows = npad // NS
    mesh = plsc.VectorSubcoreMesh(core_axis_name="c", subcore_axis_name="s")

    @functools.partial(
        pl.kernel,
        out_type=jax.ShapeDtypeStruct((NC, npad, LANES), jnp.float32),
        mesh=mesh,
        scratch_types=[
            pltpu.VMEM_SHARED((npad, LANES), jnp.float32),
            pltpu.VMEM((2, UP, IDXW), jnp.int32),
            pltpu.VMEM((2, UP, IDXW), jnp.int32),
            pltpu.VMEM((2, UP, IDXW, LANES), jnp.float32),
            pltpu.VMEM((IDXW, LANES), jnp.float32),
            pltpu.SemaphoreType.DMA,
            pltpu.SemaphoreType.DMA,
            pltpu.SemaphoreType.DMA,
            pltpu.SemaphoreType.DMA,
        ],
        compiler_params=pltpu.CompilerParams(use_tc_tiling_on_sc=False),
    )
    def prop_kernel(edge_hbm, y_hbm, out_hbm, s_acc, row_v, col_v,
                    msgs_v, zbuf, gsem0, gsem1, ssem0, ssem1):
        cid = lax.axis_index("c")
        sid = lax.axis_index("s")
        start, q = _tile_quota(cid, sid, total_rows, FRAC_C0)
        gsems = (gsem0, gsem1)
        ssems = (ssem0, ssem1)

        def zfill(i, _):
            zbuf[i] = jnp.zeros((LANES,), jnp.float32)
            return 0

        lax.fori_loop(0, IDXW, zfill, 0)
        for k in range(zrows // IDXW):
            pltpu.sync_copy(
                zbuf, s_acc.at[pl.ds(sid * zrows + k * IDXW, IDXW)]
            )
        plsc.subcore_barrier()

        gmax = total_rows - UP

        def load_fire(g, p):
            pltpu.sync_copy(edge_hbm.at[0, pl.ds(g, UP)], row_v.at[p])
            pltpu.sync_copy(edge_hbm.at[1, pl.ds(g, UP)], col_v.at[p])
            for u in range(UP):
                pltpu.async_copy(y_hbm.at[row_v.at[p, u]], msgs_v.at[p, u],
                                 gsems[p])

        def drain_g(p):
            for u in range(UP):
                pltpu.make_async_copy(
                    y_hbm.at[row_v.at[p, u]], msgs_v.at[p, u], gsems[p]
                ).wait()

        def fire_scat(p):
            for u in range(UP):
                pltpu.async_copy(msgs_v.at[p, u], s_acc.at[col_v.at[p, u]],
                                 ssems[p], add=True)

        def drain_s(p):
            for u in range(UP):
                pltpu.make_async_copy(
                    y_hbm.at[pl.ds(0, IDXW)], msgs_v.at[p, u], ssems[p]
                ).wait()

        npairs = q // (2 * UP)
        rem = q - npairs * 2 * UP
        tg = rem // UP
        tr = rem - tg * UP

        load_fire(start, 0)

        def body(tp, _):
            b = start + (2 * tp + 1) * UP
            c = jnp.minimum(start + (2 * tp + 2) * UP, gmax)
            load_fire(b, 1)
            drain_g(0)
            fire_scat(0)
            drain_g(1)
            fire_scat(1)
            drain_s(0)
            load_fire(c, 0)
            drain_s(1)
            return 0

        lax.fori_loop(0, npairs, body, 0)
        drain_g(0)

        @pl.when(tg == 1)
        def _():
            fire_scat(0)
            drain_s(0)

        def tail(i, _):
            g = start + npairs * 2 * UP + tg * UP + i
            pltpu.sync_copy(edge_hbm.at[0, pl.ds(g, 1)],
                            row_v.at[1, pl.ds(0, 1)])
            pltpu.sync_copy(edge_hbm.at[1, pl.ds(g, 1)],
                            col_v.at[1, pl.ds(0, 1)])
            pltpu.async_copy(y_hbm.at[row_v.at[1, 0]], msgs_v.at[1, 0],
                             gsem1).wait()
            pltpu.sync_copy(msgs_v.at[1, 0], s_acc.at[col_v.at[1, 0]],
                            add=True)
            return 0

        lax.fori_loop(0, tr, tail, 0)
        plsc.subcore_barrier()
        pltpu.sync_copy(
            s_acc.at[pl.ds(sid * zrows, zrows)],
            out_hbm.at[cid, pl.ds(sid * zrows, zrows)],
        )

    return prop_kernel


def _mlp_body(x_ref, w1_ref, b1_ref, w2_ref, b2_ref, p_ref, h_ref):
    h1 = lax.dot_general(
        x_ref[...], w1_ref[...], (((1,), (1,)), ((), ())),
        preferred_element_type=jnp.float32,
    ) + b1_ref[...]
    h1 = jnp.maximum(h1, 0.0) * jnp.clip(p_ref[...], 0.0, 1.0)
    h_ref[...] = lax.dot_general(
        h1, w2_ref[...], (((1,), (1,)), ((), ())),
        preferred_element_type=jnp.float32,
    ) + b2_ref[...]


def _prep_body(d0_ref, d1_ref, h_ref, dinv_ref, y0_ref):
    dinv = lax.rsqrt(d0_ref[...] + d1_ref[...] + 1.0)
    dinv_ref[...] = dinv
    y0_ref[...] = h_ref[...] * dinv


def _make_mlp(n, nfeat, nhid, nclass, bn):
    return pl.pallas_call(
        _mlp_body,
        grid=(n // bn,),
        in_specs=[
            pl.BlockSpec((bn, nfeat), lambda i: (i, 0)),
            pl.BlockSpec((nhid, nfeat), lambda i: (0, 0)),
            pl.BlockSpec((1, nhid), lambda i: (0, 0)),
            pl.BlockSpec((nclass, nhid), lambda i: (0, 0)),
            pl.BlockSpec((1, nclass), lambda i: (0, 0)),
            pl.BlockSpec((1, nhid), lambda i: (0, 0)),
        ],
        out_specs=pl.BlockSpec((bn, nclass), lambda i: (i, 0)),
        out_shape=jax.ShapeDtypeStruct((n, nclass), jnp.float32),
    )


def _make_prep(n, nclass, bn):
    spec16 = pl.BlockSpec((bn, nclass), lambda i: (i, 0))
    spec1 = pl.BlockSpec((bn, 1), lambda i: (i, 0))
    return pl.pallas_call(
        _prep_body,
        grid=(n // bn,),
        in_specs=[spec1, spec1, spec16],
        out_specs=[spec1, spec16],
        out_shape=[
            jax.ShapeDtypeStruct((n, 1), jnp.float32),
            jax.ShapeDtypeStruct((n, nclass), jnp.float32),
        ],
    )


def _make_combine(n, npad, nclass, bn, want_out, want_y):
    spec16 = pl.BlockSpec((bn, nclass), lambda i: (i, 0))
    spec1 = pl.BlockSpec((bn, 1), lambda i: (i, 0))
    spart0 = pl.BlockSpec((1, bn, nclass), lambda i: (0, i, 0))
    spart1 = pl.BlockSpec((1, bn, nclass), lambda i: (1, i, 0))

    def body(sp_ref, sp1_ref, y_ref, h_ref, dinv_ref, *outs):
        s = sp_ref[0] + sp1_ref[0] + y_ref[...]
        out = (1.0 - ALPHA) * dinv_ref[...] * s + ALPHA * h_ref[...]
        i = 0
        if want_out:
            outs[i][...] = out
            i += 1
        if want_y:
            outs[i][...] = out * dinv_ref[...]

    nouts = int(want_out) + int(want_y)
    return pl.pallas_call(
        body,
        grid=(n // bn,),
        in_specs=[spart0, spart1, spec16, spec16, spec1],
        out_specs=[spec16] * nouts,
        out_shape=[jax.ShapeDtypeStruct((n, nclass), jnp.float32)] * nouts,
    )


def kernel(x, edge_index, W1, b1, W2, b2, p):
    n, nfeat = x.shape
    nhid = W1.shape[0]
    nclass = W2.shape[0]
    e = edge_index.shape[1]

    npad = _round_up(n + 1, NS * LANES * 8)
    er = _round_up(e, IDXW)
    if er != e:
        fill = jnp.stack([
            jnp.zeros((er - e,), edge_index.dtype),
            jnp.full((er - e,), n, edge_index.dtype),
        ])
        edge_index = jnp.concatenate([edge_index, fill], axis=1)
    total_rows = er // IDXW
    edge3d = edge_index.reshape(2, total_rows, IDXW)

    degp = _make_deg_kernel(npad, total_rows)(edge3d)

    bn = 2000
    h = _make_mlp(n, nfeat, nhid, nclass, bn)(
        x, W1, b1.reshape(1, nhid), W2, b2.reshape(1, nclass),
        p.reshape(1, nhid))

    d0 = degp[0, :n].reshape(n, 1)
    d1 = degp[1, :n].reshape(n, 1)
    dinv, y0 = _make_prep(n, nclass, bn)(d0, d1, h)

    prop = _make_prop_kernel(n, npad, total_rows)
    combine_mid = _make_combine(n, npad, nclass, bn, want_out=False, want_y=True)
    combine_fin = _make_combine(n, npad, nclass, bn, want_out=True, want_y=False)

    s_part = prop(edge3d, y0)
    (y1,) = combine_mid(s_part, s_part, y0, h, dinv)
    s_part2 = prop(edge3d, y1)
    (out,) = combine_fin(s_part2, s_part2, y1, h, dinv)
    return out


# speedup vs baseline: 45.2344x; 1.0141x over previous
import functools

import jax
import jax.numpy as jnp
from jax import lax
from jax.experimental import pallas as pl
from jax.experimental.pallas import tpu as pltpu
from jax.experimental.pallas import tpu_sc as plsc

ALPHA = 0.1
K = 2

NC = 2
NS = 16
NW = NC * NS
LANES = 16
IDXW = 128
UP = 6


def _round_up(x, m):
    return (x + m - 1) // m * m


FRAC_C0 = 0.5


def _tile_quota(cid, sid, total_rows, frac0):
    q0 = int(round(total_rows * frac0))
    qc = jnp.where(cid == 0, q0, total_rows - q0)
    base_c = jnp.where(cid == 0, 0, q0)
    per = qc // NS
    ext = qc % NS
    q = per + jnp.where(sid < ext, 1, 0)
    start = base_c + sid * per + jnp.minimum(sid, ext)
    return start, q


def _make_deg_kernel(npad, total_rows):
    zrows = npad // NS
    mesh = plsc.VectorSubcoreMesh(core_axis_name="c", subcore_axis_name="s")

    @functools.partial(
        pl.kernel,
        out_type=jax.ShapeDtypeStruct((NC, npad), jnp.float32),
        mesh=mesh,
        scratch_types=[
            pltpu.VMEM_SHARED((npad,), jnp.float32),
            pltpu.VMEM((UP, IDXW), jnp.int32),
            pltpu.VMEM((IDXW,), jnp.float32),
            pltpu.VMEM((zrows,), jnp.float32),
            pltpu.SemaphoreType.DMA,
        ],
        compiler_params=pltpu.CompilerParams(use_tc_tiling_on_sc=False),
    )
    def deg_kernel(edge_hbm, out_hbm, acc, idx_v, ones_v, zeros_v, sem):
        cid = lax.axis_index("c")
        sid = lax.axis_index("s")
        start, q = _tile_quota(cid, sid, total_rows, 0.5)

        for i in range(IDXW // LANES):
            ones_v[pl.ds(i * LANES, LANES)] = jnp.ones((LANES,), jnp.float32)

        def zfill(i, _):
            zeros_v[pl.ds(i * LANES, LANES)] = jnp.zeros((LANES,), jnp.float32)
            return 0

        lax.fori_loop(0, zrows // LANES, zfill, 0)
        pltpu.sync_copy(zeros_v, acc.at[pl.ds(sid * zrows, zrows)])
        plsc.subcore_barrier()

        def body(t, _):
            g = start + t * UP
            pltpu.sync_copy(edge_hbm.at[1, pl.ds(g, UP)], idx_v)
            descs = []
            for u in range(UP):
                descs.append(
                    pltpu.async_copy(ones_v, acc.at[idx_v.at[u]], sem, add=True)
                )
            for d in descs:
                d.wait()
            return 0

        ng = q // UP
        lax.fori_loop(0, ng, body, 0)

        def tail(i, _):
            g = start + ng * UP + i
            pltpu.sync_copy(edge_hbm.at[1, pl.ds(g, 1)], idx_v.at[pl.ds(0, 1)])
            pltpu.sync_copy(ones_v, acc.at[idx_v.at[0]], add=True)
            return 0

        lax.fori_loop(0, q - ng * UP, tail, 0)
        plsc.subcore_barrier()
        pltpu.sync_copy(
            acc.at[pl.ds(sid * zrows, zrows)],
            out_hbm.at[cid, pl.ds(sid * zrows, zrows)],
        )

    return deg_kernel


def _make_prop_kernel(n, npad, total_rows):
    zrows = npad // NS
    mesh = plsc.VectorSubcoreMesh(core_axis_name="c", subcore_axis_name="s")

    @functools.partial(
        pl.kernel,
        out_type=jax.ShapeDtypeStruct((NC, npad, LANES), jnp.float32),
        mesh=mesh,
        scratch_types=[
            pltpu.VMEM_SHARED((npad, LANES), jnp.float32),
            pltpu.VMEM((2, UP, IDXW), jnp.int32),
            pltpu.VMEM((2, UP, IDXW), jnp.int32),
            pltpu.VMEM((2, UP, IDXW, LANES), jnp.float32),
            pltpu.VMEM((IDXW, LANES), jnp.float32),
            pltpu.SemaphoreType.DMA,
            pltpu.SemaphoreType.DMA,
            pltpu.SemaphoreType.DMA,
            pltpu.SemaphoreType.DMA,
        ],
        compiler_params=pltpu.CompilerParams(use_tc_tiling_on_sc=False),
    )
    def prop_kernel(edge_hbm, y_hbm, out_hbm, s_acc, row_v, col_v,
                    msgs_v, zbuf, gsem0, gsem1, ssem0, ssem1):
        cid = lax.axis_index("c")
        sid = lax.axis_index("s")
        start, q = _tile_quota(cid, sid, total_rows, FRAC_C0)
        gsems = (gsem0, gsem1)
        ssems = (ssem0, ssem1)

        def zfill(i, _):
            zbuf[i] = jnp.zeros((LANES,), jnp.float32)
            return 0

        lax.fori_loop(0, IDXW, zfill, 0)
        for k in range(zrows // IDXW):
            pltpu.sync_copy(
                zbuf, s_acc.at[pl.ds(sid * zrows + k * IDXW, IDXW)]
            )
        plsc.subcore_barrier()

        gmax = total_rows - UP

        def load_fire(g, p):
            pltpu.sync_copy(edge_hbm.at[0, pl.ds(g, UP)], row_v.at[p])
            pltpu.sync_copy(edge_hbm.at[1, pl.ds(g, UP)], col_v.at[p])
            for u in range(UP):
                pltpu.async_copy(y_hbm.at[row_v.at[p, u]], msgs_v.at[p, u],
                                 gsems[p])

        def drain_g(p):
            for u in range(UP):
                pltpu.make_async_copy(
                    y_hbm.at[row_v.at[p, u]], msgs_v.at[p, u], gsems[p]
                ).wait()

        def fire_scat(p):
            for u in range(UP):
                pltpu.async_copy(msgs_v.at[p, u], s_acc.at[col_v.at[p, u]],
                                 ssems[p], add=True)

        def drain_s(p):
            for u in range(UP):
                pltpu.make_async_copy(
                    y_hbm.at[pl.ds(0, IDXW)], msgs_v.at[p, u], ssems[p]
                ).wait()

        npairs = q // (2 * UP)
        rem = q - npairs * 2 * UP
        tg = rem // UP
        tr = rem - tg * UP

        load_fire(start, 0)

        def body(tp, _):
            b = start + (2 * tp + 1) * UP
            c = jnp.minimum(start + (2 * tp + 2) * UP, gmax)
            load_fire(b, 1)
            drain_g(0)
            fire_scat(0)
            drain_g(1)
            fire_scat(1)
            drain_s(0)
            load_fire(c, 0)
            drain_s(1)
            return 0

        lax.fori_loop(0, npairs, body, 0)
        drain_g(0)

        @pl.when(tg == 1)
        def _():
            fire_scat(0)
            drain_s(0)

        def tail(i, _):
            g = start + npairs * 2 * UP + tg * UP + i
            pltpu.sync_copy(edge_hbm.at[0, pl.ds(g, 1)],
                            row_v.at[1, pl.ds(0, 1)])
            pltpu.sync_copy(edge_hbm.at[1, pl.ds(g, 1)],
                            col_v.at[1, pl.ds(0, 1)])
            pltpu.async_copy(y_hbm.at[row_v.at[1, 0]], msgs_v.at[1, 0],
                             gsem1).wait()
            pltpu.sync_copy(msgs_v.at[1, 0], s_acc.at[col_v.at[1, 0]],
                            add=True)
            return 0

        lax.fori_loop(0, tr, tail, 0)
        plsc.subcore_barrier()
        pltpu.sync_copy(
            s_acc.at[pl.ds(sid * zrows, zrows)],
            out_hbm.at[cid, pl.ds(sid * zrows, zrows)],
        )

    return prop_kernel


def _mlp_body(x_ref, w1_ref, b1_ref, w2_ref, b2_ref, p_ref, h_ref):
    h1 = lax.dot_general(
        x_ref[...], w1_ref[...], (((1,), (1,)), ((), ())),
        preferred_element_type=jnp.float32,
    ) + b1_ref[...]
    h1 = jnp.maximum(h1, 0.0) * jnp.clip(p_ref[...], 0.0, 1.0)
    h_ref[...] = lax.dot_general(
        h1, w2_ref[...], (((1,), (1,)), ((), ())),
        preferred_element_type=jnp.float32,
    ) + b2_ref[...]


def _prep_body(d0_ref, d1_ref, h_ref, dinv_ref, y0_ref):
    dinv = lax.rsqrt(d0_ref[...] + d1_ref[...] + 1.0)
    dinv_ref[...] = dinv
    y0_ref[...] = h_ref[...] * dinv


def _make_mlp(n, nfeat, nhid, nclass, bn):
    return pl.pallas_call(
        _mlp_body,
        grid=(n // bn,),
        in_specs=[
            pl.BlockSpec((bn, nfeat), lambda i: (i, 0)),
            pl.BlockSpec((nhid, nfeat), lambda i: (0, 0)),
            pl.BlockSpec((1, nhid), lambda i: (0, 0)),
            pl.BlockSpec((nclass, nhid), lambda i: (0, 0)),
            pl.BlockSpec((1, nclass), lambda i: (0, 0)),
            pl.BlockSpec((1, nhid), lambda i: (0, 0)),
        ],
        out_specs=pl.BlockSpec((bn, nclass), lambda i: (i, 0)),
        out_shape=jax.ShapeDtypeStruct((n, nclass), jnp.float32),
    )


def _make_prep(n, nclass, bn):
    spec16 = pl.BlockSpec((bn, nclass), lambda i: (i, 0))
    spec1 = pl.BlockSpec((bn, 1), lambda i: (i, 0))
    return pl.pallas_call(
        _prep_body,
        grid=(n // bn,),
        in_specs=[spec1, spec1, spec16],
        out_specs=[spec1, spec16],
        out_shape=[
            jax.ShapeDtypeStruct((n, 1), jnp.float32),
            jax.ShapeDtypeStruct((n, nclass), jnp.float32),
        ],
    )


def _make_combine(n, npad, nclass, bn, want_out, want_y):
    spec16 = pl.BlockSpec((bn, nclass), lambda i: (i, 0))
    spec1 = pl.BlockSpec((bn, 1), lambda i: (i, 0))
    spart0 = pl.BlockSpec((1, bn, nclass), lambda i: (0, i, 0))
    spart1 = pl.BlockSpec((1, bn, nclass), lambda i: (1, i, 0))

    def body(sp_ref, sp1_ref, y_ref, h_ref, dinv_ref, *outs):
        s = sp_ref[0] + sp1_ref[0] + y_ref[...]
        out = (1.0 - ALPHA) * dinv_ref[...] * s + ALPHA * h_ref[...]
        i = 0
        if want_out:
            outs[i][...] = out
            i += 1
        if want_y:
            outs[i][...] = out * dinv_ref[...]

    nouts = int(want_out) + int(want_y)
    return pl.pallas_call(
        body,
        grid=(n // bn,),
        in_specs=[spart0, spart1, spec16, spec16, spec1],
        out_specs=[spec16] * nouts,
        out_shape=[jax.ShapeDtypeStruct((n, nclass), jnp.float32)] * nouts,
    )


def kernel(x, edge_index, W1, b1, W2, b2, p):
    n, nfeat = x.shape
    nhid = W1.shape[0]
    nclass = W2.shape[0]
    e = edge_index.shape[1]

    npad = _round_up(n + 1, NS * LANES * 8)
    er = _round_up(e, IDXW)
    if er != e:
        fill = jnp.stack([
            jnp.zeros((er - e,), edge_index.dtype),
            jnp.full((er - e,), n, edge_index.dtype),
        ])
        edge_index = jnp.concatenate([edge_index, fill], axis=1)
    total_rows = er // IDXW
    edge3d = edge_index.reshape(2, total_rows, IDXW)

    degp = _make_deg_kernel(npad, total_rows)(edge3d)

    bn = 2000
    h = _make_mlp(n, nfeat, nhid, nclass, bn)(
        x, W1, b1.reshape(1, nhid), W2, b2.reshape(1, nclass),
        p.reshape(1, nhid))

    d0 = degp[0, :n].reshape(n, 1)
    d1 = degp[1, :n].reshape(n, 1)
    dinv, y0 = _make_prep(n, nclass, bn)(d0, d1, h)

    prop = _make_prop_kernel(n, npad, total_rows)
    combine_mid = _make_combine(n, npad, nclass, bn, want_out=False, want_y=True)
    combine_fin = _make_combine(n, npad, nclass, bn, want_out=True, want_y=False)

    s_part = prop(edge3d, y0)
    (y1,) = combine_mid(s_part, s_part, y0, h, dinv)
    s_part2 = prop(edge3d, y1)
    (out,) = combine_fin(s_part2, s_part2, y1, h, dinv)
    return out
